# baseline (speedup 1.0000x reference)
import functools

import jax
import jax.numpy as jnp
from jax import lax
from jax.experimental import pallas as pl
from jax.experimental.pallas import tpu as pltpu
from jax.experimental.pallas import tpu_sc as plsc

_B, _A, _N, _T = 4, 512, 48, 256
_G = _B * _A
_NW = 32
_APW = _G // _NW
_CHUNK = 16
_NCHUNK = _APW // _CHUNK
_WPB = _A // _APW
_INV_C2 = 1.0 / 25.0
_EPS = 1e-12


def _bf16r(x):
  i = plsc.bitcast(x, jnp.int32)
  r = (i + 32768) & (-65536)
  return plsc.bitcast(r, jnp.float32)


def _body(pos_h, cell_h, an_h,
          nbr_h, mask_h, off_h,
          nbrj_h, nbrk_h, maskt_h, offj_h, offk_h,
          out_h,
          pos_v, cell_v, an_v,
          nbr_v0, mask_v0, off_v0, nbrj_v0, nbrk_v0, maskt_v0, offj_v0,
          offk_v0,
          nbr_v1, mask_v1, off_v1, nbrj_v1, nbrk_v1, maskt_v1, offj_v1,
          offk_v1,
          out_v, scr_v, sem0, sem1):
  wid = lax.axis_index("s") * 2 + lax.axis_index("c")
  batch = wid // _WPB
  albase = (wid % _WPB) * _APW
  g0 = batch * _A + albase

  hbm_chunk = (nbr_h, mask_h, off_h, nbrj_h, nbrk_h, maskt_h, offj_h, offk_h)
  bufs = (
      (nbr_v0, mask_v0, off_v0, nbrj_v0, nbrk_v0, maskt_v0, offj_v0, offk_v0),
      (nbr_v1, mask_v1, off_v1, nbrj_v1, nbrk_v1, maskt_v1, offj_v1, offk_v1),
  )
  sems = (sem0, sem1)

  def issue(cc, b):
    sl = pl.ds(g0 + cc * _CHUNK, _CHUNK)
    for src, dst in zip(hbm_chunk, bufs[b]):
      pltpu.async_copy(src.at[sl], dst, sems[b])

  def drain(b):
    sl = pl.ds(0, _CHUNK)
    for src, dst in zip(hbm_chunk, bufs[b]):
      pltpu.make_async_copy(src.at[sl], dst, sems[b]).wait()

  issue(0, 0)

  pltpu.sync_copy(pos_h, pos_v)
  pltpu.sync_copy(an_h, an_v)
  pltpu.sync_copy(cell_h, cell_v)

  ii = lax.iota(jnp.int32, 16)
  ii3 = ii * 3
  zeros = jnp.zeros((16,), jnp.float32)
  out_mask = ii < 12
  ii_clamp = jnp.minimum(ii, 11)
  gb = batch * _A
  gb3 = gb * 3
  gbv = jnp.broadcast_to(gb, (16,))
  gb3v = jnp.broadcast_to(gb3, (16,))

  cvec = _bf16r(plsc.load_gather(cell_v, [jnp.minimum(batch * 9 + ii, 35)]))
  c00 = cvec[0]
  c01 = cvec[1]
  c02 = cvec[2]
  c10 = cvec[3]
  c11 = cvec[4]
  c12 = cvec[5]
  c20 = cvec[6]
  c21 = cvec[7]
  c22 = cvec[8]

  for c in range(16):
    scr_v[pl.ds(c * 16, 16)] = zeros

  def run_chunk(cc, b):
    (nbr_v, mask_v, off_v, nbrj_v, nbrk_v, maskt_v, offj_v, offk_v) = bufs[b]

    def atom_body(ai, albase_c):
      al = albase_c + ai
      al3v = jnp.broadcast_to(gb3 + al * 3, (16,))
      aiv = jnp.broadcast_to(ai, (16,))
      pix = plsc.load_gather(pos_v, [al3v])
      piy = plsc.load_gather(pos_v, [al3v + 1])
      piz = plsc.load_gather(pos_v, [al3v + 2])
      ev = plsc.load_gather(an_v, [jnp.broadcast_to(gb + al, (16,))])

      def tri_body(t, acc):
        (a0, a1, a2, a3, a4, a5, a6, a7, a8, a9, a10, a11) = acc
        tidx = t * 16 + ii
        ij = plsc.load_gather(nbrj_v, [aiv, tidx])
        ik = plsc.load_gather(nbrk_v, [aiv, tidx])
        mt = plsc.load_gather(maskt_v, [aiv, tidx])
        oidx = ii3 + t * 48
        ojx = _bf16r(plsc.load_gather(offj_v, [aiv, oidx]))
        ojy = _bf16r(plsc.load_gather(offj_v, [aiv, oidx + 1]))
        ojz = _bf16r(plsc.load_gather(offj_v, [aiv, oidx + 2]))
        okx = _bf16r(plsc.load_gather(offk_v, [aiv, oidx]))
        oky = _bf16r(plsc.load_gather(offk_v, [aiv, oidx + 1]))
        okz = _bf16r(plsc.load_gather(offk_v, [aiv, oidx + 2]))
        ij3 = ij * 3 + gb3v
        ik3 = ik * 3 + gb3v
        pjx = plsc.load_gather(pos_v, [ij3])
        pjy = plsc.load_gather(pos_v, [ij3 + 1])
        pjz = plsc.load_gather(pos_v, [ij3 + 2])
        zja = plsc.load_gather(an_v, [ij + gbv])
        pkx = plsc.load_gather(pos_v, [ik3])
        pky = plsc.load_gather(pos_v, [ik3 + 1])
        pkz = plsc.load_gather(pos_v, [ik3 + 2])
        zka = plsc.load_gather(an_v, [ik + gbv])
        zj = zja.astype(jnp.float32)
        zk = zka.astype(jnp.float32)

        dxj = pjx + (ojx * c00 + ojy * c10 + ojz * c20) - pix
        dyj = pjy + (ojx * c01 + ojy * c11 + ojz * c21) - piy
        dzj = pjz + (ojx * c02 + ojy * c12 + ojz * c22) - piz
        dxk = pkx + (okx * c00 + oky * c10 + okz * c20) - pix
        dyk = pky + (okx * c01 + oky * c11 + okz * c21) - piy
        dzk = pkz + (okx * c02 + oky * c12 + okz * c22) - piz

        r2ij = dxj * dxj + dyj * dyj + dzj * dzj + _EPS
        r2ik = dxk * dxk + dyk * dyk + dzk * dzk + _EPS
        djk = dxj * dxk + dyj * dyk + dzj * dzk

        qj = r2ij * _INV_C2
        qk = r2ik * _INV_C2
        gq = djk + djk + _EPS
        h = gq * (0.5 * _INV_C2)
        ct2 = (gq * gq) / ((r2ij * r2ik) * 4.0)
        ang2 = 1.5 * ct2 - 0.5

        qjb = _bf16r(qj)
        qkb = _bf16r(qk)
        qj2b = _bf16r(qj * qj)
        qk2b = _bf16r(qk * qk)
        e2j = 2.0 * qjb - 1.0
        e2k = 2.0 * qkb - 1.0
        e4j = 6.0 * qj2b - 6.0 * qjb + 1.0
        e4k = 6.0 * qk2b - 6.0 * qkb + 1.0
        g4j = 4.0 * qj2b - 3.0 * qjb
        g4k = 4.0 * qk2b - 3.0 * qkb
        t3j = 3.0 * qj - 2.0
        t3k = 3.0 * qk - 2.0

        wt = mt * (zj * zk) * (1.0 / 9.0)
        wt2 = wt + wt
        wh = wt2 * h
        wth = wt * h
        wa = wt * ang2
        wa2 = wt2 * ang2

        a0 = a0 + wt2
        a1 = a1 + wt * (e2j + e2k)
        a2 = a2 + wt * (e4j + e4k)
        a3 = a3 + wt2 * (e2j * e2k)
        a4 = a4 + wt * (e2j * e4k + e2k * e4j)
        a5 = a5 + wt2 * (e4j * e4k)
        a6 = a6 + wh
        a7 = a7 + wth * (t3j + t3k)
        a8 = a8 + wh * (t3j * t3k)
        a9 = a9 + wa2 * (qjb * qkb)
        a10 = a10 + wa * (qjb * g4k + qkb * g4j)
        a11 = a11 + wa2 * (g4j * g4k)
        return (a0, a1, a2, a3, a4, a5, a6, a7, a8, a9, a10, a11)

      acc0 = (zeros,) * 12
      acc = plsc.parallel_loop(0, _T // 16, unroll=1, carry=acc0)(tri_body)

      def pair_body(p, acc):
        (a0, a1, a2, a3, a4, a5, a6, a7, a8, a9, a10, a11) = acc
        pidx = p * 16 + ii
        nj = plsc.load_gather(nbr_v, [aiv, pidx])
        mk = plsc.load_gather(mask_v, [aiv, pidx])
        oidx = ii3 + p * 48
        ox = _bf16r(plsc.load_gather(off_v, [aiv, oidx]))
        oy = _bf16r(plsc.load_gather(off_v, [aiv, oidx + 1]))
        oz = _bf16r(plsc.load_gather(off_v, [aiv, oidx + 2]))
        nj3 = nj * 3 + gb3v
        pjx = plsc.load_gather(pos_v, [nj3])
        pjy = plsc.load_gather(pos_v, [nj3 + 1])
        pjz = plsc.load_gather(pos_v, [nj3 + 2])
        zja = plsc.load_gather(an_v, [nj + gbv])
        zj = zja.astype(jnp.float32)

        dx = pjx + (ox * c00 + oy * c10 + oz * c20) - pix
        dy = pjy + (ox * c01 + oy * c11 + oz * c21) - piy
        dz = pjz + (ox * c02 + oy * c12 + oz * c22) - piz
        r2 = dx * dx + dy * dy + dz * dz + _EPS
        u = r2 * _INV_C2
        ub = _bf16r(u)
        u2b = _bf16r(u * u)
        e2 = 2.0 * ub - 1.0
        e4 = 6.0 * u2b - 6.0 * ub + 1.0
        g4 = 4.0 * u2b - 3.0 * ub
        t3 = 3.0 * u - 2.0

        w = (mk + mk) * (zj * zj) * (1.0 / 9.0)
        wu = w * u
        wg = w * g4
        a0 = a0 + w
        a1 = a1 + w * e2
        a2 = a2 + w * e4
        a3 = a3 + w * (e2 * e2)
        a4 = a4 + w * (e2 * e4)
        a5 = a5 + w * (e4 * e4)
        a6 = a6 + wu
        a7 = a7 + wu * t3
        a8 = a8 + wu * (t3 * t3)
        a9 = a9 + w * (ub * ub)
        a10 = a10 + w * (ub * g4)
        a11 = a11 + wg * g4
        return (a0, a1, a2, a3, a4, a5, a6, a7, a8, a9, a10, a11)

      acc = plsc.parallel_loop(0, _N // 16, unroll=1, carry=acc)(pair_body)

      for c in range(12):
        scr_v[pl.ds(c * 16, 16)] = acc[c]

      def red_body(l, s):
        return s + plsc.load_gather(scr_v, [ii * 16 + l])

      s = plsc.parallel_loop(0, 16, unroll=4, carry=zeros)(red_body)

      idxs = ev * 12 + ii_clamp
      plsc.store_scatter(out_v, [aiv, idxs], s, mask=out_mask)
      return albase_c

    for ai in range(_CHUNK):
      for sseg in range(3):
        out_v[ai, pl.ds(sseg * 16, 16)] = zeros
    lax.fori_loop(0, _CHUNK, atom_body, albase + cc * _CHUNK)
    pltpu.sync_copy(out_v, out_h.at[pl.ds(g0 + cc * _CHUNK, _CHUNK)])

  for cc in range(_NCHUNK):
    b = cc % 2
    drain(b)
    if cc + 1 < _NCHUNK:
      issue(cc + 1, 1 - b)
    run_chunk(cc, b)


@functools.cache
def _build_sc_kernel():
  mesh = plsc.VectorSubcoreMesh(
      core_axis_name="c", subcore_axis_name="s", num_cores=2, num_subcores=16)
  chunk_bufs = [
      pltpu.VMEM((_CHUNK, _N), jnp.int32),
      pltpu.VMEM((_CHUNK, _N), jnp.float32),
      pltpu.VMEM((_CHUNK, _N * 3), jnp.float32),
      pltpu.VMEM((_CHUNK, _T), jnp.int32),
      pltpu.VMEM((_CHUNK, _T), jnp.int32),
      pltpu.VMEM((_CHUNK, _T), jnp.float32),
      pltpu.VMEM((_CHUNK, _T * 3), jnp.float32),
      pltpu.VMEM((_CHUNK, _T * 3), jnp.float32),
  ]
  return functools.partial(
      pl.kernel,
      out_type=jax.ShapeDtypeStruct((_G, 48), jnp.float32),
      mesh=mesh,
      compiler_params=pltpu.CompilerParams(needs_layout_passes=False),
      scratch_types=[
          pltpu.VMEM((_G * 3,), jnp.float32),
          pltpu.VMEM((_B * 9,), jnp.float32),
          pltpu.VMEM((_G,), jnp.int32),
      ] + chunk_bufs + chunk_bufs + [
          pltpu.VMEM((_CHUNK, 48), jnp.float32),
          pltpu.VMEM((256,), jnp.float32),
          pltpu.SemaphoreType.DMA,
          pltpu.SemaphoreType.DMA,
      ],
  )(_body)


@jax.jit
def kernel(positions, cell, neighbors, neighbors_j, neighbors_k, mask,
           mask_triples, offsets, offsets_j, offsets_k, atomic_numbers):
  out = _build_sc_kernel()(
      positions.reshape(_G * 3).astype(jnp.float32),
      cell.reshape(_B * 9).astype(jnp.float32),
      atomic_numbers.reshape(_G).astype(jnp.int32),
      neighbors.reshape(_G, _N).astype(jnp.int32),
      mask.reshape(_G, _N).astype(jnp.float32),
      offsets.reshape(_G, _N * 3).astype(jnp.float32),
      neighbors_j.reshape(_G, _T).astype(jnp.int32),
      neighbors_k.reshape(_G, _T).astype(jnp.int32),
      mask_triples.reshape(_G, _T).astype(jnp.float32),
      offsets_j.reshape(_G, _T * 3).astype(jnp.float32),
      offsets_k.reshape(_G, _T * 3).astype(jnp.float32))
  return out.reshape(_B, _A, 48)


# speedup vs baseline: 1.0808x; 1.0808x over previous
import functools

import jax
import jax.numpy as jnp
from jax import lax
from jax.experimental import pallas as pl
from jax.experimental.pallas import tpu as pltpu
from jax.experimental.pallas import tpu_sc as plsc

_B, _A, _N, _T = 4, 512, 48, 256
_G = _B * _A
_NW = 32
_APW = _G // _NW
_CHUNK = 16
_NCHUNK = _APW // _CHUNK
_WPB = _A // _APW
_INV_C2 = 1.0 / 25.0
_EPS = 1e-12


def _bf16r(x):
  i = plsc.bitcast(x, jnp.int32)
  r = (i + 32768) & (-65536)
  return plsc.bitcast(r, jnp.float32)


def _body(pos_h, cell_h, an_h,
          nbr_h, mask_h, off_h,
          nbrj_h, nbrk_h, maskt_h, offj_h, offk_h,
          out_h,
          pos_v, cell_v, an_v,
          nbr_v0, mask_v0, off_v0, nbrj_v0, nbrk_v0, maskt_v0, offj_v0,
          offk_v0,
          nbr_v1, mask_v1, off_v1, nbrj_v1, nbrk_v1, maskt_v1, offj_v1,
          offk_v1,
          out_v, scr_v, sem0, sem1):
  wid = lax.axis_index("s") * 2 + lax.axis_index("c")
  batch = wid // _WPB
  albase = (wid % _WPB) * _APW
  g0 = batch * _A + albase

  hbm_chunk = (nbr_h, mask_h, off_h, nbrj_h, nbrk_h, maskt_h, offj_h, offk_h)
  bufs = (
      (nbr_v0, mask_v0, off_v0, nbrj_v0, nbrk_v0, maskt_v0, offj_v0, offk_v0),
      (nbr_v1, mask_v1, off_v1, nbrj_v1, nbrk_v1, maskt_v1, offj_v1, offk_v1),
  )
  sems = (sem0, sem1)

  def issue(cc, b):
    sl = pl.ds(g0 + cc * _CHUNK, _CHUNK)
    for src, dst in zip(hbm_chunk, bufs[b]):
      pltpu.async_copy(src.at[sl], dst, sems[b])

  def drain(b):
    sl = pl.ds(0, _CHUNK)
    for src, dst in zip(hbm_chunk, bufs[b]):
      pltpu.make_async_copy(src.at[sl], dst, sems[b]).wait()

  issue(0, 0)

  pltpu.sync_copy(pos_h, pos_v)
  pltpu.sync_copy(an_h, an_v)
  pltpu.sync_copy(cell_h, cell_v)

  ii = lax.iota(jnp.int32, 16)
  ii3 = ii * 3
  zeros = jnp.zeros((16,), jnp.float32)
  out_mask = ii < 12
  ii_clamp = jnp.minimum(ii, 11)
  gb = batch * _A
  gb3 = gb * 3
  gbv = jnp.broadcast_to(gb, (16,))
  gb3v = jnp.broadcast_to(gb3, (16,))

  cvec = _bf16r(plsc.load_gather(cell_v, [jnp.minimum(batch * 9 + ii, 35)]))
  c00 = cvec[0]
  c01 = cvec[1]
  c02 = cvec[2]
  c10 = cvec[3]
  c11 = cvec[4]
  c12 = cvec[5]
  c20 = cvec[6]
  c21 = cvec[7]
  c22 = cvec[8]

  for c in range(16):
    scr_v[pl.ds(c * 16, 16)] = zeros

  def run_chunk(cc, b):
    (nbr_v, mask_v, off_v, nbrj_v, nbrk_v, maskt_v, offj_v, offk_v) = bufs[b]

    def atom_body(ai, albase_c):
      al = albase_c + ai
      al3v = jnp.broadcast_to(gb3 + al * 3, (16,))
      aiv = jnp.broadcast_to(ai, (16,))
      pix = plsc.load_gather(pos_v, [al3v])
      piy = plsc.load_gather(pos_v, [al3v + 1])
      piz = plsc.load_gather(pos_v, [al3v + 2])
      ev = plsc.load_gather(an_v, [jnp.broadcast_to(gb + al, (16,))])

      def tri_body(t, acc):
        (a0, a1, a2, a3, a4, a5, a6, a7, a8, a9, a10, a11) = acc
        tidx = t * 16 + ii
        ij = plsc.load_gather(nbrj_v, [aiv, tidx])
        ik = plsc.load_gather(nbrk_v, [aiv, tidx])
        mt = plsc.load_gather(maskt_v, [aiv, tidx])
        oidx = ii3 + t * 48
        ojx = _bf16r(plsc.load_gather(offj_v, [aiv, oidx]))
        ojy = _bf16r(plsc.load_gather(offj_v, [aiv, oidx + 1]))
        ojz = _bf16r(plsc.load_gather(offj_v, [aiv, oidx + 2]))
        okx = _bf16r(plsc.load_gather(offk_v, [aiv, oidx]))
        oky = _bf16r(plsc.load_gather(offk_v, [aiv, oidx + 1]))
        okz = _bf16r(plsc.load_gather(offk_v, [aiv, oidx + 2]))
        ij3 = ij * 3 + gb3v
        ik3 = ik * 3 + gb3v
        pjx = plsc.load_gather(pos_v, [ij3])
        pjy = plsc.load_gather(pos_v, [ij3 + 1])
        pjz = plsc.load_gather(pos_v, [ij3 + 2])
        zja = plsc.load_gather(an_v, [ij + gbv])
        pkx = plsc.load_gather(pos_v, [ik3])
        pky = plsc.load_gather(pos_v, [ik3 + 1])
        pkz = plsc.load_gather(pos_v, [ik3 + 2])
        zka = plsc.load_gather(an_v, [ik + gbv])
        zj = zja.astype(jnp.float32)
        zk = zka.astype(jnp.float32)

        dxj = pjx + (ojx * c00 + ojy * c10 + ojz * c20) - pix
        dyj = pjy + (ojx * c01 + ojy * c11 + ojz * c21) - piy
        dzj = pjz + (ojx * c02 + ojy * c12 + ojz * c22) - piz
        dxk = pkx + (okx * c00 + oky * c10 + okz * c20) - pix
        dyk = pky + (okx * c01 + oky * c11 + okz * c21) - piy
        dzk = pkz + (okx * c02 + oky * c12 + okz * c22) - piz

        r2ij = dxj * dxj + dyj * dyj + dzj * dzj + _EPS
        r2ik = dxk * dxk + dyk * dyk + dzk * dzk + _EPS
        djk = dxj * dxk + dyj * dyk + dzj * dzk

        qj = r2ij * _INV_C2
        qk = r2ik * _INV_C2
        gq = djk + djk + _EPS
        h = gq * (0.5 * _INV_C2)
        ct2 = (gq * gq) / ((r2ij * r2ik) * 4.0)
        ang2 = 1.5 * ct2 - 0.5

        qjb = _bf16r(qj)
        qkb = _bf16r(qk)
        qj2b = _bf16r(qj * qj)
        qk2b = _bf16r(qk * qk)
        s1 = qjb + qkb
        s2 = qj2b + qk2b
        p11 = qjb * qkb
        p22 = qj2b * qk2b
        p12 = qjb * qk2b + qkb * qj2b
        sq = qj + qk
        pq = qj * qk

        wt = mt * (zj * zk) * (1.0 / 9.0)
        wth = wt * h
        wh = wth + wth
        wa = wt * ang2

        a0 = a0 + wt
        a1 = a1 + wt * s1
        a2 = a2 + wt * s2
        a3 = a3 + wt * p11
        a4 = a4 + wt * p12
        a5 = a5 + wt * p22
        a6 = a6 + wh
        a7 = a7 + wth * sq
        a8 = a8 + wh * pq
        a9 = a9 + wa * p11
        a10 = a10 + wa * p12
        a11 = a11 + wa * p22
        return (a0, a1, a2, a3, a4, a5, a6, a7, a8, a9, a10, a11)

      acc0 = (zeros,) * 12
      mom = plsc.parallel_loop(0, _T // 16, unroll=2, carry=acc0)(tri_body)
      (m00, m01, m02, m11, m12, m22, a6m, r01, r11, n11, n12, n22) = mom
      acc = (
          m00 + m00,
          2.0 * (m01 - m00),
          6.0 * m02 - 6.0 * m01 + 2.0 * m00,
          8.0 * m11 - 4.0 * m01 + 2.0 * m00,
          12.0 * m12 - 24.0 * m11 - 6.0 * m02 + 8.0 * m01 - 2.0 * m00,
          72.0 * m22 - 72.0 * m12 + 72.0 * m11 + 12.0 * m02 - 12.0 * m01
          + 2.0 * m00,
          a6m,
          3.0 * r01 - 2.0 * a6m,
          9.0 * r11 - 12.0 * r01 + 4.0 * a6m,
          n11 + n11,
          4.0 * n12 - 6.0 * n11,
          32.0 * n22 - 24.0 * n12 + 18.0 * n11,
      )

      def pair_body(p, acc):
        (a0, a1, a2, a3, a4, a5, a6, a7, a8, a9, a10, a11) = acc
        pidx = p * 16 + ii
        nj = plsc.load_gather(nbr_v, [aiv, pidx])
        mk = plsc.load_gather(mask_v, [aiv, pidx])
        oidx = ii3 + p * 48
        ox = _bf16r(plsc.load_gather(off_v, [aiv, oidx]))
        oy = _bf16r(plsc.load_gather(off_v, [aiv, oidx + 1]))
        oz = _bf16r(plsc.load_gather(off_v, [aiv, oidx + 2]))
        nj3 = nj * 3 + gb3v
        pjx = plsc.load_gather(pos_v, [nj3])
        pjy = plsc.load_gather(pos_v, [nj3 + 1])
        pjz = plsc.load_gather(pos_v, [nj3 + 2])
        zja = plsc.load_gather(an_v, [nj + gbv])
        zj = zja.astype(jnp.float32)

        dx = pjx + (ox * c00 + oy * c10 + oz * c20) - pix
        dy = pjy + (ox * c01 + oy * c11 + oz * c21) - piy
        dz = pjz + (ox * c02 + oy * c12 + oz * c22) - piz
        r2 = dx * dx + dy * dy + dz * dz + _EPS
        u = r2 * _INV_C2
        ub = _bf16r(u)
        u2b = _bf16r(u * u)
        e2 = 2.0 * ub - 1.0
        e4 = 6.0 * u2b - 6.0 * ub + 1.0
        g4 = 4.0 * u2b - 3.0 * ub
        t3 = 3.0 * u - 2.0

        w = (mk + mk) * (zj * zj) * (1.0 / 9.0)
        wu = w * u
        wg = w * g4
        a0 = a0 + w
        a1 = a1 + w * e2
        a2 = a2 + w * e4
        a3 = a3 + w * (e2 * e2)
        a4 = a4 + w * (e2 * e4)
        a5 = a5 + w * (e4 * e4)
        a6 = a6 + wu
        a7 = a7 + wu * t3
        a8 = a8 + wu * (t3 * t3)
        a9 = a9 + w * (ub * ub)
        a10 = a10 + w * (ub * g4)
        a11 = a11 + wg * g4
        return (a0, a1, a2, a3, a4, a5, a6, a7, a8, a9, a10, a11)

      acc = plsc.parallel_loop(0, _N // 16, unroll=2, carry=acc)(pair_body)

      for c in range(12):
        scr_v[pl.ds(c * 16, 16)] = acc[c]

      def red_body(l, s):
        return s + plsc.load_gather(scr_v, [ii * 16 + l])

      s = plsc.parallel_loop(0, 16, unroll=4, carry=zeros)(red_body)

      idxs = ev * 12 + ii_clamp
      plsc.store_scatter(out_v, [aiv, idxs], s, mask=out_mask)
      return albase_c

    for ai in range(_CHUNK):
      for sseg in range(3):
        out_v[ai, pl.ds(sseg * 16, 16)] = zeros
    lax.fori_loop(0, _CHUNK, atom_body, albase + cc * _CHUNK)
    pltpu.sync_copy(out_v, out_h.at[pl.ds(g0 + cc * _CHUNK, _CHUNK)])

  for cc in range(_NCHUNK):
    b = cc % 2
    drain(b)
    if cc + 1 < _NCHUNK:
      issue(cc + 1, 1 - b)
    run_chunk(cc, b)


@functools.cache
def _build_sc_kernel():
  mesh = plsc.VectorSubcoreMesh(
      core_axis_name="c", subcore_axis_name="s", num_cores=2, num_subcores=16)
  chunk_bufs = [
      pltpu.VMEM((_CHUNK, _N), jnp.int32),
      pltpu.VMEM((_CHUNK, _N), jnp.float32),
      pltpu.VMEM((_CHUNK, _N * 3), jnp.float32),
      pltpu.VMEM((_CHUNK, _T), jnp.int32),
      pltpu.VMEM((_CHUNK, _T), jnp.int32),
      pltpu.VMEM((_CHUNK, _T), jnp.float32),
      pltpu.VMEM((_CHUNK, _T * 3), jnp.float32),
      pltpu.VMEM((_CHUNK, _T * 3), jnp.float32),
  ]
  return functools.partial(
      pl.kernel,
      out_type=jax.ShapeDtypeStruct((_G, 48), jnp.float32),
      mesh=mesh,
      compiler_params=pltpu.CompilerParams(needs_layout_passes=False),
      scratch_types=[
          pltpu.VMEM((_G * 3,), jnp.float32),
          pltpu.VMEM((_B * 9,), jnp.float32),
          pltpu.VMEM((_G,), jnp.int32),
      ] + chunk_bufs + chunk_bufs + [
          pltpu.VMEM((_CHUNK, 48), jnp.float32),
          pltpu.VMEM((256,), jnp.float32),
          pltpu.SemaphoreType.DMA,
          pltpu.SemaphoreType.DMA,
      ],
  )(_body)


@jax.jit
def kernel(positions, cell, neighbors, neighbors_j, neighbors_k, mask,
           mask_triples, offsets, offsets_j, offsets_k, atomic_numbers):
  out = _build_sc_kernel()(
      positions.reshape(_G * 3).astype(jnp.float32),
      cell.reshape(_B * 9).astype(jnp.float32),
      atomic_numbers.reshape(_G).astype(jnp.int32),
      neighbors.reshape(_G, _N).astype(jnp.int32),
      mask.reshape(_G, _N).astype(jnp.float32),
      offsets.reshape(_G, _N * 3).astype(jnp.float32),
      neighbors_j.reshape(_G, _T).astype(jnp.int32),
      neighbors_k.reshape(_G, _T).astype(jnp.int32),
      mask_triples.reshape(_G, _T).astype(jnp.float32),
      offsets_j.reshape(_G, _T * 3).astype(jnp.float32),
      offsets_k.reshape(_G, _T * 3).astype(jnp.float32))
  return out.reshape(_B, _A, 48)
